# Initial kernel scaffold; baseline (speedup 1.0000x reference)
import jax
import jax.numpy as jnp
from jax.experimental import pallas as pl


def kernel(x, edge_index, edge_attr, solvent_fingerprint, batch, W1, b1, g1, be1, W2, b2, g2, be2, Ws, bs, Wf1, bf1, gf1, bef1, Wf2, bf2):
    raise NotImplementedError("write your pallas kernel here")


# speedup vs baseline: 7.6419x; 7.6419x over previous
import jax
import jax.numpy as jnp
from jax import lax
from jax.experimental import pallas as pl
from jax.experimental.pallas import tpu as pltpu
from jax.experimental.pallas import tpu_sc as plsc

N_NODES = 10000
N_PAD = 10240
E_EDGES = 320000
CHUNKS32 = 80
CHUNKS16 = 160
E_PAD = 32 * CHUNKS32 * 128
G = 256
G_PAD = 384
ROWS_PT = N_PAD // 16
GROWS_PT = G_PAD // 16
IDX_BLK = 16
EPS = 1e-5
F32 = jnp.float32


def _sc_mesh():
    return plsc.VectorSubcoreMesh(core_axis_name="c", subcore_axis_name="s")


def _deg_body(dst3, zeros_hbm, ones_hbm, out_hbm, dst_v, ones_v, acc_sh):
    c = lax.axis_index("c")
    s = lax.axis_index("s")
    wid = c * 16 + s
    pltpu.sync_copy(zeros_hbm.at[pl.ds(s * ROWS_PT, ROWS_PT)],
                    acc_sh.at[pl.ds(s * ROWS_PT, ROWS_PT)])
    pltpu.sync_copy(ones_hbm, ones_v)
    plsc.subcore_barrier()

    def block(b, carry):
        pltpu.sync_copy(dst3.at[wid, pl.ds(b * IDX_BLK, IDX_BLK)], dst_v)

        def body(j, carry2):
            pltpu.sync_copy(ones_v, acc_sh.at[dst_v.at[j]], add=True)
            return carry2

        lax.fori_loop(0, IDX_BLK, body, 0)
        return carry

    lax.fori_loop(0, CHUNKS32 // IDX_BLK, block, 0)
    plsc.subcore_barrier()
    pltpu.sync_copy(acc_sh.at[pl.ds(s * ROWS_PT, ROWS_PT)],
                    out_hbm.at[c, pl.ds(s * ROWS_PT, ROWS_PT)])


def _degree(dst32):
    zeros = jnp.zeros((N_PAD, 128), F32)
    ones = jnp.ones((128, 128), F32)
    f = pl.kernel(
        _deg_body,
        out_type=jax.ShapeDtypeStruct((2, N_PAD, 128), F32),
        mesh=_sc_mesh(),
        scratch_types=[
            pltpu.VMEM((IDX_BLK, 128), jnp.int32),
            pltpu.VMEM((128, 128), F32),
            pltpu.VMEM_SHARED((N_PAD, 128), F32),
        ],
    )
    return f(dst32, zeros, ones)


def _edge_sweep(y, acc_sh, src3, dst3, wid, n_chunks,
                src_v, dst_v, buf0, buf1, sem0, sem1):
    def block(b, carry):
        pltpu.sync_copy(src3.at[wid, pl.ds(b * IDX_BLK, IDX_BLK)], src_v)
        pltpu.sync_copy(dst3.at[wid, pl.ds(b * IDX_BLK, IDX_BLK)], dst_v)

        def body(jj, carry2):
            j0 = jj * 2
            j1 = j0 + 1
            cp0 = pltpu.async_copy(y.at[src_v.at[j0]], buf0, sem0)
            cp1 = pltpu.async_copy(y.at[src_v.at[j1]], buf1, sem1)
            cp0.wait()
            pltpu.sync_copy(buf0, acc_sh.at[dst_v.at[j0]], add=True)
            cp1.wait()
            pltpu.sync_copy(buf1, acc_sh.at[dst_v.at[j1]], add=True)
            return carry2

        lax.fori_loop(0, IDX_BLK // 2, body, 0)
        return carry

    lax.fori_loop(0, n_chunks // IDX_BLK, block, 0)


def _agg1_body(y, src3, dst3, out_hbm, src_v, dst_v, buf0, buf1,
               acc_sh, sem0, sem1):
    c = lax.axis_index("c")
    s = lax.axis_index("s")
    wid = c * 16 + s
    pltpu.sync_copy(y.at[pl.ds(s * ROWS_PT, ROWS_PT)],
                    acc_sh.at[pl.ds(s * ROWS_PT, ROWS_PT)])
    plsc.subcore_barrier()
    _edge_sweep(y, acc_sh, src3, dst3, wid, CHUNKS32,
                src_v, dst_v, buf0, buf1, sem0, sem1)
    plsc.subcore_barrier()
    pltpu.sync_copy(acc_sh.at[pl.ds(s * ROWS_PT, ROWS_PT)],
                    out_hbm.at[c, pl.ds(s * ROWS_PT, ROWS_PT)])


def _aggregate1(y, src32, dst32):
    f = pl.kernel(
        _agg1_body,
        out_type=jax.ShapeDtypeStruct((2, N_PAD, 128), F32),
        mesh=_sc_mesh(),
        scratch_types=[
            pltpu.VMEM((IDX_BLK, 128), jnp.int32),
            pltpu.VMEM((IDX_BLK, 128), jnp.int32),
            pltpu.VMEM((128, 128), F32),
            pltpu.VMEM((128, 128), F32),
            pltpu.VMEM_SHARED((N_PAD, 128), F32),
            pltpu.SemaphoreType.DMA,
            pltpu.SemaphoreType.DMA,
        ],
    )
    return f(y, src32, dst32)


def _agg2_body(y0, y1, src3, dst3, o0, o1,
               src_v, dst_v, buf0, buf1, acc_sh, sem0, sem1):
    c = lax.axis_index("c")
    s = lax.axis_index("s")

    def run(y, o):
        pltpu.sync_copy(y.at[pl.ds(s * ROWS_PT, ROWS_PT)],
                        acc_sh.at[pl.ds(s * ROWS_PT, ROWS_PT)])
        plsc.subcore_barrier()
        _edge_sweep(y, acc_sh, src3, dst3, s, CHUNKS16,
                    src_v, dst_v, buf0, buf1, sem0, sem1)
        plsc.subcore_barrier()
        pltpu.sync_copy(acc_sh.at[pl.ds(s * ROWS_PT, ROWS_PT)],
                        o.at[pl.ds(s * ROWS_PT, ROWS_PT)])

    @pl.when(c == 0)
    def _():
        run(y0, o0)

    @pl.when(c == 1)
    def _():
        run(y1, o1)


def _aggregate2(y0, y1, src16, dst16):
    f = pl.kernel(
        _agg2_body,
        out_type=(jax.ShapeDtypeStruct((N_PAD, 128), F32),
                  jax.ShapeDtypeStruct((N_PAD, 128), F32)),
        mesh=_sc_mesh(),
        scratch_types=[
            pltpu.VMEM((IDX_BLK, 128), jnp.int32),
            pltpu.VMEM((IDX_BLK, 128), jnp.int32),
            pltpu.VMEM((128, 128), F32),
            pltpu.VMEM((128, 128), F32),
            pltpu.VMEM_SHARED((N_PAD, 128), F32),
            pltpu.SemaphoreType.DMA,
            pltpu.SemaphoreType.DMA,
        ],
    )
    return f(y0, y1, src16, dst16)


def _pool_body(h0, h1, b3, zsum_hbm, osum, hbuf, bidx, acc_sh):
    c = lax.axis_index("c")
    s = lax.axis_index("s")
    pltpu.sync_copy(b3.at[s], bidx)

    def run(h, o):
        pltpu.sync_copy(zsum_hbm.at[pl.ds(s * GROWS_PT, GROWS_PT)],
                        acc_sh.at[pl.ds(s * GROWS_PT, GROWS_PT)])
        pltpu.sync_copy(h.at[pl.ds(s * ROWS_PT, ROWS_PT)], hbuf)
        plsc.subcore_barrier()
        for k in range(5):
            pltpu.sync_copy(hbuf.at[pl.ds(k * 128, 128)],
                            acc_sh.at[bidx.at[k]], add=True)
        plsc.subcore_barrier()
        pltpu.sync_copy(acc_sh.at[pl.ds(s * GROWS_PT, GROWS_PT)],
                        o.at[pl.ds(s * GROWS_PT, GROWS_PT)])

    @pl.when(c == 0)
    def _():
        run(h0, osum.at[0])

    @pl.when(c == 1)
    def _():
        run(h1, osum.at[1])


def _pool(h0, h1, batch3):
    zsum = jnp.zeros((G_PAD, 128), F32)
    f = pl.kernel(
        _pool_body,
        out_type=jax.ShapeDtypeStruct((2, G_PAD, 128), F32),
        mesh=_sc_mesh(),
        scratch_types=[
            pltpu.VMEM((ROWS_PT, 128), F32),
            pltpu.VMEM((5, 128), jnp.int32),
            pltpu.VMEM_SHARED((G_PAD, 128), F32),
        ],
    )
    return f(h0, h1, batch3, zsum)


def _dinv_col(degw_val):
    deg = degw_val[0, :, 0:1] + degw_val[1, :, 0:1] + 1.0
    return lax.rsqrt(deg)


def _mm1_body(x_ref, w_ref, degw_ref, y_ref):
    dinv = _dinv_col(degw_ref[...])
    y_ref[...] = jnp.dot(x_ref[...], w_ref[...],
                         preferred_element_type=F32) * dinv


def _mm1(x_p, w1, degw):
    return pl.pallas_call(
        _mm1_body,
        out_shape=jax.ShapeDtypeStruct((N_PAD, 128), F32),
    )(x_p, w1, degw)


def _masked_bn_relu(t, g, be):
    rid = lax.broadcasted_iota(jnp.int32, (N_PAD, 1), 0)
    valid = rid < N_NODES
    tm = jnp.where(valid, t, 0.0)
    mean = jnp.sum(tm, axis=0, keepdims=True) / N_NODES
    dv = jnp.where(valid, t - mean, 0.0)
    var = jnp.sum(dv * dv, axis=0, keepdims=True) / N_NODES
    h = jax.nn.relu(g * (t - mean) * lax.rsqrt(var + EPS) + be)
    return jnp.where(valid, h, 0.0)


def _bn_mm2_body(p, y, degw, b1r, g1r, be1r, w2, y0_ref, y1_ref):
    dinv = _dinv_col(degw[...])
    agg = p[0] + p[1] - y[...]
    t = agg * dinv + b1r[...]
    h = _masked_bn_relu(t, g1r[...], be1r[...])
    y2 = jnp.dot(h, w2[...], preferred_element_type=F32) * dinv
    y0_ref[...] = y2[:, :128]
    y1_ref[...] = y2[:, 128:]


def _bn_mm2(p, y, degw, b1, g1, be1, w2):
    return pl.pallas_call(
        _bn_mm2_body,
        out_shape=(jax.ShapeDtypeStruct((N_PAD, 128), F32),
                   jax.ShapeDtypeStruct((N_PAD, 128), F32)),
    )(p, y, degw, b1.reshape(1, -1), g1.reshape(1, -1),
      be1.reshape(1, -1), w2)


def _bn2_body(a0, a1, degw, b2r, g2r, be2r, h0_ref, h1_ref):
    dinv = _dinv_col(degw[...])
    t = jnp.concatenate([a0[...], a1[...]], axis=1) * dinv + b2r[...]
    h = _masked_bn_relu(t, g2r[...], be2r[...])
    h0_ref[...] = h[:, :128]
    h1_ref[...] = h[:, 128:]


def _bn2(a0, a1, degw, b2, g2, be2):
    return pl.pallas_call(
        _bn2_body,
        out_shape=(jax.ShapeDtypeStruct((N_PAD, 128), F32),
                   jax.ShapeDtypeStruct((N_PAD, 128), F32)),
    )(a0, a1, degw, b2.reshape(1, -1), g2.reshape(1, -1), be2.reshape(1, -1))


def _head_body(sum_ref, batch_ref, sfp, ws, bsr, wf1, bf1r, gf1r, bef1r,
               wf2, bf2r, out_ref):
    sums = jnp.concatenate([sum_ref[0, :G, :], sum_ref[1, :G, :]], axis=1)
    gid = lax.broadcasted_iota(jnp.int32, (G, N_PAD), 0)
    onehot = (gid == batch_ref[...]).astype(F32)
    cnt = jnp.sum(onehot, axis=1, keepdims=True)
    pooled = sums / jnp.maximum(cnt, 1.0)
    solv = jax.nn.relu(
        jnp.dot(sfp[...], ws[...], preferred_element_type=F32) + bsr[...])
    z = jnp.concatenate([pooled, solv], axis=1)
    u = jnp.dot(z, wf1[...], preferred_element_type=F32) + bf1r[...]
    m = jnp.mean(u, axis=0, keepdims=True)
    d = u - m
    v = jnp.mean(d * d, axis=0, keepdims=True)
    z2 = jax.nn.relu(gf1r[...] * d * lax.rsqrt(v + EPS) + bef1r[...])
    out_ref[...] = jnp.dot(z2, wf2[...], preferred_element_type=F32) + bf2r[...]


def _head(sums, batch_row, sfp, ws, bs, wf1, bf1, gf1, bef1, wf2, bf2):
    return pl.pallas_call(
        _head_body,
        out_shape=jax.ShapeDtypeStruct((G, 1), F32),
    )(sums, batch_row, sfp, ws, bs.reshape(1, -1), wf1, bf1.reshape(1, -1),
      gf1.reshape(1, -1), bef1.reshape(1, -1), wf2, bf2.reshape(1, -1))


def kernel(x, edge_index, edge_attr, solvent_fingerprint, batch,
           W1, b1, g1, be1, W2, b2, g2, be2, Ws, bs,
           Wf1, bf1, gf1, bef1, Wf2, bf2):
    del edge_attr
    x = x.astype(F32)
    src = edge_index[0].astype(jnp.int32)
    dst = edge_index[1].astype(jnp.int32)
    pad_e = E_PAD - E_EDGES
    src_p = jnp.concatenate([src, jnp.full((pad_e,), N_NODES, jnp.int32)])
    dst_p = jnp.concatenate([dst, jnp.full((pad_e,), N_NODES, jnp.int32)])
    src32 = src_p.reshape(32, CHUNKS32, 128)
    dst32 = dst_p.reshape(32, CHUNKS32, 128)
    src16 = src_p.reshape(16, CHUNKS16, 128)
    dst16 = dst_p.reshape(16, CHUNKS16, 128)
    batch3 = jnp.concatenate(
        [batch.astype(jnp.int32),
         jnp.full((N_PAD - N_NODES,), G, jnp.int32)]).reshape(16, 5, 128)
    x_p = jnp.pad(x, ((0, N_PAD - N_NODES), (0, 0)))

    degw = _degree(dst32)
    y = _mm1(x_p, W1, degw)
    p = _aggregate1(y, src32, dst32)
    y2_0, y2_1 = _bn_mm2(p, y, degw, b1, g1, be1, W2)
    c0, c1 = _aggregate2(y2_0, y2_1, src16, dst16)
    h0, h1 = _bn2(c0, c1, degw, b2, g2, be2)
    sums = _pool(h0, h1, batch3)
    batch_row = batch3.reshape(1, N_PAD)
    return _head(sums, batch_row, solvent_fingerprint.astype(F32),
                 Ws, bs, Wf1, bf1, gf1, bef1, Wf2, bf2)


# speedup vs baseline: 7.6614x; 1.0026x over previous
import jax
import jax.numpy as jnp
from jax import lax
from jax.experimental import pallas as pl
from jax.experimental.pallas import tpu as pltpu
from jax.experimental.pallas import tpu_sc as plsc

N_NODES = 10000
N_PAD = 10240
E_EDGES = 320000
CHUNKS32 = 80
CHUNKS16 = 160
E_PAD = 32 * CHUNKS32 * 128
G = 256
G_PAD = 384
ROWS_PT = N_PAD // 16
GROWS_PT = G_PAD // 16
IDX_BLK = 16
EPS = 1e-5
F32 = jnp.float32


def _sc_mesh():
    return plsc.VectorSubcoreMesh(core_axis_name="c", subcore_axis_name="s")


def _deg_body(dst3, zeros_hbm, ones_hbm, out_hbm, dst_v, ones_v, acc_sh):
    c = lax.axis_index("c")
    s = lax.axis_index("s")
    wid = c * 16 + s
    pltpu.sync_copy(zeros_hbm.at[pl.ds(s * ROWS_PT, ROWS_PT)],
                    acc_sh.at[pl.ds(s * ROWS_PT, ROWS_PT)])
    pltpu.sync_copy(ones_hbm, ones_v)
    plsc.subcore_barrier()

    def block(b, carry):
        pltpu.sync_copy(dst3.at[wid, pl.ds(b * IDX_BLK, IDX_BLK)], dst_v)

        def body(j, carry2):
            pltpu.sync_copy(ones_v, acc_sh.at[dst_v.at[j]], add=True)
            return carry2

        lax.fori_loop(0, IDX_BLK, body, 0)
        return carry

    lax.fori_loop(0, CHUNKS32 // IDX_BLK, block, 0)
    plsc.subcore_barrier()
    pltpu.sync_copy(acc_sh.at[pl.ds(s * ROWS_PT, ROWS_PT)],
                    out_hbm.at[c, pl.ds(s * ROWS_PT, ROWS_PT)])


def _degree(dst32):
    zeros = jnp.zeros((N_PAD, 128), F32)
    ones = jnp.ones((128, 128), F32)
    f = pl.kernel(
        _deg_body,
        out_type=jax.ShapeDtypeStruct((2, N_PAD, 128), F32),
        mesh=_sc_mesh(),
        scratch_types=[
            pltpu.VMEM((IDX_BLK, 128), jnp.int32),
            pltpu.VMEM((128, 128), F32),
            pltpu.VMEM_SHARED((N_PAD, 128), F32),
        ],
    )
    return f(dst32, zeros, ones)


def _edge_sweep(y, acc_sh, src3, dst3, wid, n_chunks,
                src_v, dst_v, buf0, buf1, sem0, sem1, ssem0, ssem1):
    def block(b, carry):
        pltpu.sync_copy(src3.at[wid, pl.ds(b * IDX_BLK, IDX_BLK)], src_v)
        pltpu.sync_copy(dst3.at[wid, pl.ds(b * IDX_BLK, IDX_BLK)], dst_v)

        def body(jj, carry2):
            j0 = jj * 2
            j1 = j0 + 1
            cp0 = pltpu.async_copy(y.at[src_v.at[j0]], buf0, sem0)
            cp1 = pltpu.async_copy(y.at[src_v.at[j1]], buf1, sem1)
            cp0.wait()
            sc0 = pltpu.async_copy(buf0, acc_sh.at[dst_v.at[j0]], ssem0,
                                   add=True)
            cp1.wait()
            sc1 = pltpu.async_copy(buf1, acc_sh.at[dst_v.at[j1]], ssem1,
                                   add=True)
            sc0.wait()
            sc1.wait()
            return carry2

        lax.fori_loop(0, IDX_BLK // 2, body, 0)
        return carry

    lax.fori_loop(0, n_chunks // IDX_BLK, block, 0)


def _agg1_body(y, src3, dst3, out_hbm, src_v, dst_v, buf0, buf1,
               acc_sh, sem0, sem1, ssem0, ssem1):
    c = lax.axis_index("c")
    s = lax.axis_index("s")
    wid = c * 16 + s
    pltpu.sync_copy(y.at[pl.ds(s * ROWS_PT, ROWS_PT)],
                    acc_sh.at[pl.ds(s * ROWS_PT, ROWS_PT)])
    plsc.subcore_barrier()
    _edge_sweep(y, acc_sh, src3, dst3, wid, CHUNKS32,
                src_v, dst_v, buf0, buf1, sem0, sem1, ssem0, ssem1)
    plsc.subcore_barrier()
    pltpu.sync_copy(acc_sh.at[pl.ds(s * ROWS_PT, ROWS_PT)],
                    out_hbm.at[c, pl.ds(s * ROWS_PT, ROWS_PT)])


def _aggregate1(y, src32, dst32):
    f = pl.kernel(
        _agg1_body,
        out_type=jax.ShapeDtypeStruct((2, N_PAD, 128), F32),
        mesh=_sc_mesh(),
        scratch_types=[
            pltpu.VMEM((IDX_BLK, 128), jnp.int32),
            pltpu.VMEM((IDX_BLK, 128), jnp.int32),
            pltpu.VMEM((128, 128), F32),
            pltpu.VMEM((128, 128), F32),
            pltpu.VMEM_SHARED((N_PAD, 128), F32),
            pltpu.SemaphoreType.DMA,
            pltpu.SemaphoreType.DMA,
            pltpu.SemaphoreType.DMA,
            pltpu.SemaphoreType.DMA,
        ],
    )
    return f(y, src32, dst32)


def _agg2_body(y0, y1, src3, dst3, o0, o1,
               src_v, dst_v, buf0, buf1, acc_sh, sem0, sem1, ssem0, ssem1):
    c = lax.axis_index("c")
    s = lax.axis_index("s")

    def run(y, o):
        pltpu.sync_copy(y.at[pl.ds(s * ROWS_PT, ROWS_PT)],
                        acc_sh.at[pl.ds(s * ROWS_PT, ROWS_PT)])
        plsc.subcore_barrier()
        _edge_sweep(y, acc_sh, src3, dst3, s, CHUNKS16,
                    src_v, dst_v, buf0, buf1, sem0, sem1, ssem0, ssem1)
        plsc.subcore_barrier()
        pltpu.sync_copy(acc_sh.at[pl.ds(s * ROWS_PT, ROWS_PT)],
                        o.at[pl.ds(s * ROWS_PT, ROWS_PT)])

    @pl.when(c == 0)
    def _():
        run(y0, o0)

    @pl.when(c == 1)
    def _():
        run(y1, o1)


def _aggregate2(y0, y1, src16, dst16):
    f = pl.kernel(
        _agg2_body,
        out_type=(jax.ShapeDtypeStruct((N_PAD, 128), F32),
                  jax.ShapeDtypeStruct((N_PAD, 128), F32)),
        mesh=_sc_mesh(),
        scratch_types=[
            pltpu.VMEM((IDX_BLK, 128), jnp.int32),
            pltpu.VMEM((IDX_BLK, 128), jnp.int32),
            pltpu.VMEM((128, 128), F32),
            pltpu.VMEM((128, 128), F32),
            pltpu.VMEM_SHARED((N_PAD, 128), F32),
            pltpu.SemaphoreType.DMA,
            pltpu.SemaphoreType.DMA,
            pltpu.SemaphoreType.DMA,
            pltpu.SemaphoreType.DMA,
        ],
    )
    return f(y0, y1, src16, dst16)


def _pool_body(h0, h1, b3, zsum_hbm, osum, hbuf, bidx, acc_sh):
    c = lax.axis_index("c")
    s = lax.axis_index("s")
    pltpu.sync_copy(b3.at[s], bidx)

    def run(h, o):
        pltpu.sync_copy(zsum_hbm.at[pl.ds(s * GROWS_PT, GROWS_PT)],
                        acc_sh.at[pl.ds(s * GROWS_PT, GROWS_PT)])
        pltpu.sync_copy(h.at[pl.ds(s * ROWS_PT, ROWS_PT)], hbuf)
        plsc.subcore_barrier()
        for k in range(5):
            pltpu.sync_copy(hbuf.at[pl.ds(k * 128, 128)],
                            acc_sh.at[bidx.at[k]], add=True)
        plsc.subcore_barrier()
        pltpu.sync_copy(acc_sh.at[pl.ds(s * GROWS_PT, GROWS_PT)],
                        o.at[pl.ds(s * GROWS_PT, GROWS_PT)])

    @pl.when(c == 0)
    def _():
        run(h0, osum.at[0])

    @pl.when(c == 1)
    def _():
        run(h1, osum.at[1])


def _pool(h0, h1, batch3):
    zsum = jnp.zeros((G_PAD, 128), F32)
    f = pl.kernel(
        _pool_body,
        out_type=jax.ShapeDtypeStruct((2, G_PAD, 128), F32),
        mesh=_sc_mesh(),
        scratch_types=[
            pltpu.VMEM((ROWS_PT, 128), F32),
            pltpu.VMEM((5, 128), jnp.int32),
            pltpu.VMEM_SHARED((G_PAD, 128), F32),
        ],
    )
    return f(h0, h1, batch3, zsum)


def _dinv_col(degw_val):
    deg = degw_val[0, :, 0:1] + degw_val[1, :, 0:1] + 1.0
    return lax.rsqrt(deg)


def _mm1_body(x_ref, w_ref, degw_ref, y_ref):
    dinv = _dinv_col(degw_ref[...])
    y_ref[...] = jnp.dot(x_ref[...], w_ref[...],
                         preferred_element_type=F32) * dinv


def _mm1(x_p, w1, degw):
    return pl.pallas_call(
        _mm1_body,
        out_shape=jax.ShapeDtypeStruct((N_PAD, 128), F32),
    )(x_p, w1, degw)


def _masked_bn_relu(t, g, be):
    rid = lax.broadcasted_iota(jnp.int32, (N_PAD, 1), 0)
    valid = rid < N_NODES
    tm = jnp.where(valid, t, 0.0)
    mean = jnp.sum(tm, axis=0, keepdims=True) / N_NODES
    dv = jnp.where(valid, t - mean, 0.0)
    var = jnp.sum(dv * dv, axis=0, keepdims=True) / N_NODES
    h = jax.nn.relu(g * (t - mean) * lax.rsqrt(var + EPS) + be)
    return jnp.where(valid, h, 0.0)


def _bn_mm2_body(p, y, degw, b1r, g1r, be1r, w2, y0_ref, y1_ref):
    dinv = _dinv_col(degw[...])
    agg = p[0] + p[1] - y[...]
    t = agg * dinv + b1r[...]
    h = _masked_bn_relu(t, g1r[...], be1r[...])
    y2 = jnp.dot(h, w2[...], preferred_element_type=F32) * dinv
    y0_ref[...] = y2[:, :128]
    y1_ref[...] = y2[:, 128:]


def _bn_mm2(p, y, degw, b1, g1, be1, w2):
    return pl.pallas_call(
        _bn_mm2_body,
        out_shape=(jax.ShapeDtypeStruct((N_PAD, 128), F32),
                   jax.ShapeDtypeStruct((N_PAD, 128), F32)),
    )(p, y, degw, b1.reshape(1, -1), g1.reshape(1, -1),
      be1.reshape(1, -1), w2)


def _bn2_body(a0, a1, degw, b2r, g2r, be2r, h0_ref, h1_ref):
    dinv = _dinv_col(degw[...])
    t = jnp.concatenate([a0[...], a1[...]], axis=1) * dinv + b2r[...]
    h = _masked_bn_relu(t, g2r[...], be2r[...])
    h0_ref[...] = h[:, :128]
    h1_ref[...] = h[:, 128:]


def _bn2(a0, a1, degw, b2, g2, be2):
    return pl.pallas_call(
        _bn2_body,
        out_shape=(jax.ShapeDtypeStruct((N_PAD, 128), F32),
                   jax.ShapeDtypeStruct((N_PAD, 128), F32)),
    )(a0, a1, degw, b2.reshape(1, -1), g2.reshape(1, -1), be2.reshape(1, -1))


def _head_body(sum_ref, batch_ref, sfp, ws, bsr, wf1, bf1r, gf1r, bef1r,
               wf2, bf2r, out_ref):
    sums = jnp.concatenate([sum_ref[0, :G, :], sum_ref[1, :G, :]], axis=1)
    gid = lax.broadcasted_iota(jnp.int32, (G, N_PAD), 0)
    onehot = (gid == batch_ref[...]).astype(F32)
    cnt = jnp.sum(onehot, axis=1, keepdims=True)
    pooled = sums / jnp.maximum(cnt, 1.0)
    solv = jax.nn.relu(
        jnp.dot(sfp[...], ws[...], preferred_element_type=F32) + bsr[...])
    z = jnp.concatenate([pooled, solv], axis=1)
    u = jnp.dot(z, wf1[...], preferred_element_type=F32) + bf1r[...]
    m = jnp.mean(u, axis=0, keepdims=True)
    d = u - m
    v = jnp.mean(d * d, axis=0, keepdims=True)
    z2 = jax.nn.relu(gf1r[...] * d * lax.rsqrt(v + EPS) + bef1r[...])
    out_ref[...] = jnp.dot(z2, wf2[...], preferred_element_type=F32) + bf2r[...]


def _head(sums, batch_row, sfp, ws, bs, wf1, bf1, gf1, bef1, wf2, bf2):
    return pl.pallas_call(
        _head_body,
        out_shape=jax.ShapeDtypeStruct((G, 1), F32),
    )(sums, batch_row, sfp, ws, bs.reshape(1, -1), wf1, bf1.reshape(1, -1),
      gf1.reshape(1, -1), bef1.reshape(1, -1), wf2, bf2.reshape(1, -1))


def kernel(x, edge_index, edge_attr, solvent_fingerprint, batch,
           W1, b1, g1, be1, W2, b2, g2, be2, Ws, bs,
           Wf1, bf1, gf1, bef1, Wf2, bf2):
    del edge_attr
    x = x.astype(F32)
    src = edge_index[0].astype(jnp.int32)
    dst = edge_index[1].astype(jnp.int32)
    pad_e = E_PAD - E_EDGES
    src_p = jnp.concatenate([src, jnp.full((pad_e,), N_NODES, jnp.int32)])
    dst_p = jnp.concatenate([dst, jnp.full((pad_e,), N_NODES, jnp.int32)])
    src32 = src_p.reshape(32, CHUNKS32, 128)
    dst32 = dst_p.reshape(32, CHUNKS32, 128)
    src16 = src_p.reshape(16, CHUNKS16, 128)
    dst16 = dst_p.reshape(16, CHUNKS16, 128)
    batch3 = jnp.concatenate(
        [batch.astype(jnp.int32),
         jnp.full((N_PAD - N_NODES,), G, jnp.int32)]).reshape(16, 5, 128)
    x_p = jnp.pad(x, ((0, N_PAD - N_NODES), (0, 0)))

    degw = _degree(dst32)
    y = _mm1(x_p, W1, degw)
    p = _aggregate1(y, src32, dst32)
    y2_0, y2_1 = _bn_mm2(p, y, degw, b1, g1, be1, W2)
    c0, c1 = _aggregate2(y2_0, y2_1, src16, dst16)
    h0, h1 = _bn2(c0, c1, degw, b2, g2, be2)
    sums = _pool(h0, h1, batch3)
    batch_row = batch3.reshape(1, N_PAD)
    return _head(sums, batch_row, solvent_fingerprint.astype(F32),
                 Ws, bs, Wf1, bf1, gf1, bef1, Wf2, bf2)


# speedup vs baseline: 8.1141x; 1.0591x over previous
import jax
import jax.numpy as jnp
from jax import lax
from jax.experimental import pallas as pl
from jax.experimental.pallas import tpu as pltpu
from jax.experimental.pallas import tpu_sc as plsc

N_NODES = 10000
N_PAD = 10240
E_EDGES = 320000
CHUNKS32 = 80
CHUNKS16 = 160
E_PAD = 32 * CHUNKS32 * 128
G = 256
G_PAD = 384
ROWS_PT = N_PAD // 16
GROWS_PT = G_PAD // 16
IDX_BLK = 40
EPS = 1e-5
F32 = jnp.float32


def _sc_mesh():
    return plsc.VectorSubcoreMesh(core_axis_name="c", subcore_axis_name="s")


def _deg_body(dst3, zeros_hbm, ones_hbm, out_hbm, dst_v, ones_v, acc_sh):
    c = lax.axis_index("c")
    s = lax.axis_index("s")
    wid = c * 16 + s
    pltpu.sync_copy(zeros_hbm.at[pl.ds(s * ROWS_PT, ROWS_PT)],
                    acc_sh.at[pl.ds(s * ROWS_PT, ROWS_PT)])
    pltpu.sync_copy(ones_hbm, ones_v)
    plsc.subcore_barrier()

    def block(b, carry):
        pltpu.sync_copy(dst3.at[wid, pl.ds(b * IDX_BLK, IDX_BLK)], dst_v)

        def body(j, carry2):
            pltpu.sync_copy(ones_v, acc_sh.at[dst_v.at[j]], add=True)
            return carry2

        lax.fori_loop(0, IDX_BLK, body, 0)
        return carry

    lax.fori_loop(0, CHUNKS32 // IDX_BLK, block, 0)
    plsc.subcore_barrier()
    pltpu.sync_copy(acc_sh.at[pl.ds(s * ROWS_PT, ROWS_PT)],
                    out_hbm.at[c, pl.ds(s * ROWS_PT, ROWS_PT)])


def _degree(dst32):
    zeros = jnp.zeros((N_PAD, 128), F32)
    ones = jnp.ones((128, 128), F32)
    f = pl.kernel(
        _deg_body,
        out_type=jax.ShapeDtypeStruct((2, N_PAD, 128), F32),
        mesh=_sc_mesh(),
        scratch_types=[
            pltpu.VMEM((IDX_BLK, 128), jnp.int32),
            pltpu.VMEM((128, 128), F32),
            pltpu.VMEM_SHARED((N_PAD, 128), F32),
        ],
    )
    return f(dst32, zeros, ones)


def _edge_sweep(y, acc_sh, src3, dst3, wid, n_chunks,
                src_v, dst_v, buf0, buf1, sem0, sem1, ssem0, ssem1):
    def drain(buf, sem):
        pltpu.make_async_copy(y.at[pl.ds(0, 128)], buf, sem).wait()

    def block(b, carry):
        pltpu.sync_copy(src3.at[wid, pl.ds(b * IDX_BLK, IDX_BLK)], src_v)
        pltpu.sync_copy(dst3.at[wid, pl.ds(b * IDX_BLK, IDX_BLK)], dst_v)

        def body(jj, carry2):
            j0 = jj * 2
            j1 = j0 + 1
            first = jnp.logical_and(b == 0, jj == 0)

            @pl.when(jnp.logical_not(first))
            def _():
                drain(buf0, ssem0)

            cp0 = pltpu.async_copy(y.at[src_v.at[j0]], buf0, sem0)

            @pl.when(jnp.logical_not(first))
            def _():
                drain(buf1, ssem1)

            cp1 = pltpu.async_copy(y.at[src_v.at[j1]], buf1, sem1)
            cp0.wait()
            pltpu.async_copy(buf0, acc_sh.at[dst_v.at[j0]], ssem0, add=True)
            cp1.wait()
            pltpu.async_copy(buf1, acc_sh.at[dst_v.at[j1]], ssem1, add=True)
            return carry2

        lax.fori_loop(0, IDX_BLK // 2, body, 0)
        return carry

    lax.fori_loop(0, n_chunks // IDX_BLK, block, 0)
    drain(buf0, ssem0)
    drain(buf1, ssem1)


def _agg1_body(y, src3, dst3, out_hbm, src_v, dst_v, buf0, buf1,
               acc_sh, sem0, sem1, ssem0, ssem1):
    c = lax.axis_index("c")
    s = lax.axis_index("s")
    wid = c * 16 + s
    pltpu.sync_copy(y.at[pl.ds(s * ROWS_PT, ROWS_PT)],
                    acc_sh.at[pl.ds(s * ROWS_PT, ROWS_PT)])
    plsc.subcore_barrier()
    _edge_sweep(y, acc_sh, src3, dst3, wid, CHUNKS32,
                src_v, dst_v, buf0, buf1, sem0, sem1, ssem0, ssem1)
    plsc.subcore_barrier()
    pltpu.sync_copy(acc_sh.at[pl.ds(s * ROWS_PT, ROWS_PT)],
                    out_hbm.at[c, pl.ds(s * ROWS_PT, ROWS_PT)])


def _aggregate1(y, src32, dst32):
    f = pl.kernel(
        _agg1_body,
        out_type=jax.ShapeDtypeStruct((2, N_PAD, 128), F32),
        mesh=_sc_mesh(),
        scratch_types=[
            pltpu.VMEM((IDX_BLK, 128), jnp.int32),
            pltpu.VMEM((IDX_BLK, 128), jnp.int32),
            pltpu.VMEM((128, 128), F32),
            pltpu.VMEM((128, 128), F32),
            pltpu.VMEM_SHARED((N_PAD, 128), F32),
            pltpu.SemaphoreType.DMA,
            pltpu.SemaphoreType.DMA,
            pltpu.SemaphoreType.DMA,
            pltpu.SemaphoreType.DMA,
        ],
    )
    return f(y, src32, dst32)


def _agg2_body(y0, y1, src3, dst3, o0, o1,
               src_v, dst_v, buf0, buf1, acc_sh, sem0, sem1, ssem0, ssem1):
    c = lax.axis_index("c")
    s = lax.axis_index("s")

    def run(y, o):
        pltpu.sync_copy(y.at[pl.ds(s * ROWS_PT, ROWS_PT)],
                        acc_sh.at[pl.ds(s * ROWS_PT, ROWS_PT)])
        plsc.subcore_barrier()
        _edge_sweep(y, acc_sh, src3, dst3, s, CHUNKS16,
                    src_v, dst_v, buf0, buf1, sem0, sem1, ssem0, ssem1)
        plsc.subcore_barrier()
        pltpu.sync_copy(acc_sh.at[pl.ds(s * ROWS_PT, ROWS_PT)],
                        o.at[pl.ds(s * ROWS_PT, ROWS_PT)])

    @pl.when(c == 0)
    def _():
        run(y0, o0)

    @pl.when(c == 1)
    def _():
        run(y1, o1)


def _aggregate2(y0, y1, src16, dst16):
    f = pl.kernel(
        _agg2_body,
        out_type=(jax.ShapeDtypeStruct((N_PAD, 128), F32),
                  jax.ShapeDtypeStruct((N_PAD, 128), F32)),
        mesh=_sc_mesh(),
        scratch_types=[
            pltpu.VMEM((IDX_BLK, 128), jnp.int32),
            pltpu.VMEM((IDX_BLK, 128), jnp.int32),
            pltpu.VMEM((128, 128), F32),
            pltpu.VMEM((128, 128), F32),
            pltpu.VMEM_SHARED((N_PAD, 128), F32),
            pltpu.SemaphoreType.DMA,
            pltpu.SemaphoreType.DMA,
            pltpu.SemaphoreType.DMA,
            pltpu.SemaphoreType.DMA,
        ],
    )
    return f(y0, y1, src16, dst16)


def _pool_body(h0, h1, b3, zsum_hbm, osum, hbuf, bidx, acc_sh):
    c = lax.axis_index("c")
    s = lax.axis_index("s")
    pltpu.sync_copy(b3.at[s], bidx)

    def run(h, o):
        pltpu.sync_copy(zsum_hbm.at[pl.ds(s * GROWS_PT, GROWS_PT)],
                        acc_sh.at[pl.ds(s * GROWS_PT, GROWS_PT)])
        pltpu.sync_copy(h.at[pl.ds(s * ROWS_PT, ROWS_PT)], hbuf)
        plsc.subcore_barrier()
        for k in range(5):
            pltpu.sync_copy(hbuf.at[pl.ds(k * 128, 128)],
                            acc_sh.at[bidx.at[k]], add=True)
        plsc.subcore_barrier()
        pltpu.sync_copy(acc_sh.at[pl.ds(s * GROWS_PT, GROWS_PT)],
                        o.at[pl.ds(s * GROWS_PT, GROWS_PT)])

    @pl.when(c == 0)
    def _():
        run(h0, osum.at[0])

    @pl.when(c == 1)
    def _():
        run(h1, osum.at[1])


def _pool(h0, h1, batch3):
    zsum = jnp.zeros((G_PAD, 128), F32)
    f = pl.kernel(
        _pool_body,
        out_type=jax.ShapeDtypeStruct((2, G_PAD, 128), F32),
        mesh=_sc_mesh(),
        scratch_types=[
            pltpu.VMEM((ROWS_PT, 128), F32),
            pltpu.VMEM((5, 128), jnp.int32),
            pltpu.VMEM_SHARED((G_PAD, 128), F32),
        ],
    )
    return f(h0, h1, batch3, zsum)


def _dinv_col(degw_val):
    deg = degw_val[0, :, 0:1] + degw_val[1, :, 0:1] + 1.0
    return lax.rsqrt(deg)


def _mm1_body(x_ref, w_ref, degw_ref, y_ref):
    dinv = _dinv_col(degw_ref[...])
    y_ref[...] = jnp.dot(x_ref[...], w_ref[...],
                         preferred_element_type=F32) * dinv


def _mm1(x_p, w1, degw):
    return pl.pallas_call(
        _mm1_body,
        out_shape=jax.ShapeDtypeStruct((N_PAD, 128), F32),
    )(x_p, w1, degw)


def _masked_bn_relu(t, g, be):
    rid = lax.broadcasted_iota(jnp.int32, (N_PAD, 1), 0)
    valid = rid < N_NODES
    tm = jnp.where(valid, t, 0.0)
    mean = jnp.sum(tm, axis=0, keepdims=True) / N_NODES
    dv = jnp.where(valid, t - mean, 0.0)
    var = jnp.sum(dv * dv, axis=0, keepdims=True) / N_NODES
    h = jax.nn.relu(g * (t - mean) * lax.rsqrt(var + EPS) + be)
    return jnp.where(valid, h, 0.0)


def _bn_mm2_body(p, y, degw, b1r, g1r, be1r, w2, y0_ref, y1_ref):
    dinv = _dinv_col(degw[...])
    agg = p[0] + p[1] - y[...]
    t = agg * dinv + b1r[...]
    h = _masked_bn_relu(t, g1r[...], be1r[...])
    y2 = jnp.dot(h, w2[...], preferred_element_type=F32) * dinv
    y0_ref[...] = y2[:, :128]
    y1_ref[...] = y2[:, 128:]


def _bn_mm2(p, y, degw, b1, g1, be1, w2):
    return pl.pallas_call(
        _bn_mm2_body,
        out_shape=(jax.ShapeDtypeStruct((N_PAD, 128), F32),
                   jax.ShapeDtypeStruct((N_PAD, 128), F32)),
    )(p, y, degw, b1.reshape(1, -1), g1.reshape(1, -1),
      be1.reshape(1, -1), w2)


def _bn2_body(a0, a1, degw, b2r, g2r, be2r, h0_ref, h1_ref):
    dinv = _dinv_col(degw[...])
    t = jnp.concatenate([a0[...], a1[...]], axis=1) * dinv + b2r[...]
    h = _masked_bn_relu(t, g2r[...], be2r[...])
    h0_ref[...] = h[:, :128]
    h1_ref[...] = h[:, 128:]


def _bn2(a0, a1, degw, b2, g2, be2):
    return pl.pallas_call(
        _bn2_body,
        out_shape=(jax.ShapeDtypeStruct((N_PAD, 128), F32),
                   jax.ShapeDtypeStruct((N_PAD, 128), F32)),
    )(a0, a1, degw, b2.reshape(1, -1), g2.reshape(1, -1), be2.reshape(1, -1))


def _head_body(sum_ref, batch_ref, sfp, ws, bsr, wf1, bf1r, gf1r, bef1r,
               wf2, bf2r, out_ref):
    sums = jnp.concatenate([sum_ref[0, :G, :], sum_ref[1, :G, :]], axis=1)
    gid = lax.broadcasted_iota(jnp.int32, (G, N_PAD), 0)
    onehot = (gid == batch_ref[...]).astype(F32)
    cnt = jnp.sum(onehot, axis=1, keepdims=True)
    pooled = sums / jnp.maximum(cnt, 1.0)
    solv = jax.nn.relu(
        jnp.dot(sfp[...], ws[...], preferred_element_type=F32) + bsr[...])
    z = jnp.concatenate([pooled, solv], axis=1)
    u = jnp.dot(z, wf1[...], preferred_element_type=F32) + bf1r[...]
    m = jnp.mean(u, axis=0, keepdims=True)
    d = u - m
    v = jnp.mean(d * d, axis=0, keepdims=True)
    z2 = jax.nn.relu(gf1r[...] * d * lax.rsqrt(v + EPS) + bef1r[...])
    out_ref[...] = jnp.dot(z2, wf2[...], preferred_element_type=F32) + bf2r[...]


def _head(sums, batch_row, sfp, ws, bs, wf1, bf1, gf1, bef1, wf2, bf2):
    return pl.pallas_call(
        _head_body,
        out_shape=jax.ShapeDtypeStruct((G, 1), F32),
    )(sums, batch_row, sfp, ws, bs.reshape(1, -1), wf1, bf1.reshape(1, -1),
      gf1.reshape(1, -1), bef1.reshape(1, -1), wf2, bf2.reshape(1, -1))


def kernel(x, edge_index, edge_attr, solvent_fingerprint, batch,
           W1, b1, g1, be1, W2, b2, g2, be2, Ws, bs,
           Wf1, bf1, gf1, bef1, Wf2, bf2):
    del edge_attr
    x = x.astype(F32)
    src = edge_index[0].astype(jnp.int32)
    dst = edge_index[1].astype(jnp.int32)
    pad_e = E_PAD - E_EDGES
    src_p = jnp.concatenate([src, jnp.full((pad_e,), N_NODES, jnp.int32)])
    dst_p = jnp.concatenate([dst, jnp.full((pad_e,), N_NODES, jnp.int32)])
    src32 = src_p.reshape(32, CHUNKS32, 128)
    dst32 = dst_p.reshape(32, CHUNKS32, 128)
    src16 = src_p.reshape(16, CHUNKS16, 128)
    dst16 = dst_p.reshape(16, CHUNKS16, 128)
    batch3 = jnp.concatenate(
        [batch.astype(jnp.int32),
         jnp.full((N_PAD - N_NODES,), G, jnp.int32)]).reshape(16, 5, 128)
    x_p = jnp.pad(x, ((0, N_PAD - N_NODES), (0, 0)))

    degw = _degree(dst32)
    y = _mm1(x_p, W1, degw)
    p = _aggregate1(y, src32, dst32)
    y2_0, y2_1 = _bn_mm2(p, y, degw, b1, g1, be1, W2)
    c0, c1 = _aggregate2(y2_0, y2_1, src16, dst16)
    h0, h1 = _bn2(c0, c1, degw, b2, g2, be2)
    sums = _pool(h0, h1, batch3)
    batch_row = batch3.reshape(1, N_PAD)
    return _head(sums, batch_row, solvent_fingerprint.astype(F32),
                 Ws, bs, Wf1, bf1, gf1, bef1, Wf2, bf2)
